# baseline (speedup 1.0000x reference)
import jax
import jax.numpy as jnp
from jax.experimental import pallas as pl

B, T, L = 8, 32, 64
D = 768
H = 12
dh = D // H
TT = 32
NT = T // TT


def _body(spk_row_ref, spk_col_ref, tok_ref, edu_ref, wqt_ref, wkt_ref,
          wvt_ref, out_ref):
    f32 = jnp.float32
    bf16 = jnp.bfloat16
    i = pl.program_id(1)
    t0 = i * TT

    tok = tok_ref[0]
    q = jax.lax.dot(tok.astype(bf16), wqt_ref[:].astype(bf16),
                    preferred_element_type=f32).astype(bf16)
    edu = edu_ref[0]
    k = jax.lax.dot(edu.astype(bf16), wkt_ref[:].astype(bf16),
                    preferred_element_type=f32).astype(bf16)
    v = jax.lax.dot(edu.astype(bf16), wvt_ref[:].astype(bf16),
                    preferred_element_type=f32).astype(bf16)

    spk_row = spk_row_ref[0]
    spk_col = spk_col_ref[0]
    same = spk_col == spk_row
    trow = jax.lax.broadcasted_iota(jnp.int32, (TT, T), 0) + t0
    ucol = jax.lax.broadcasted_iota(jnp.int32, (TT, T), 1)
    mask = (same & (ucol <= trow)).astype(f32)
    mask = mask.reshape(TT, 1, T)

    parts = []
    for h in range(H):
        sl = slice(h * dh, (h + 1) * dh)
        q_h = q[:, sl]
        k_h = k[:, sl]
        v_h = v[:, sl]
        s = jax.lax.dot_general(q_h, k_h, (((1,), (1,)), ((), ())),
                                preferred_element_type=f32)
        s = (s.reshape(TT, L, T) * mask).astype(bf16)
        a_h = jax.lax.dot(s.reshape(TT * L, T), v_h,
                          preferred_element_type=f32)
        parts.append(a_h)
    out_ref[0] = tok + jnp.concatenate(parts, axis=1)


def kernel(input_ids, speaker_names, token_embeddings, edu_embeddings,
           Wk, Wv, Wq):
    tok = token_embeddings.reshape(B, T * L, D)
    spk = speaker_names.astype(jnp.int32)
    spk_row = spk.reshape(B, 1, T)
    spk_col = spk.reshape(B, T, 1)

    out = pl.pallas_call(
        _body,
        grid=(B, NT),
        in_specs=[
            pl.BlockSpec((1, 1, T), lambda b, i: (b, 0, 0)),
            pl.BlockSpec((1, TT, 1), lambda b, i: (b, i, 0)),
            pl.BlockSpec((1, TT * L, D), lambda b, i: (b, i, 0)),
            pl.BlockSpec((1, T, D), lambda b, i: (b, 0, 0)),
            pl.BlockSpec((D, D), lambda b, i: (0, 0)),
            pl.BlockSpec((D, D), lambda b, i: (0, 0)),
            pl.BlockSpec((D, D), lambda b, i: (0, 0)),
        ],
        out_specs=pl.BlockSpec((1, TT * L, D), lambda b, i: (b, i, 0)),
        out_shape=jax.ShapeDtypeStruct((B, T * L, D), jnp.float32),
    )(spk_row, spk_col, tok, edu_embeddings, Wq.T, Wk.T, Wv.T)
    return out.reshape(B, T, L, D)


# speedup vs baseline: 1.0724x; 1.0724x over previous
import jax
import jax.numpy as jnp
from jax.experimental import pallas as pl
from jax.experimental.pallas import tpu as pltpu

B, T, L = 8, 32, 64
D = 768
H = 12
dh = D // H
TT = 16
NT = T // TT


def _body(spk_row_ref, spk_col_ref, tok_ref, edu_ref, wqt_ref, wkt_ref,
          wvt_ref, out_ref, wq_scr, k_scr, v_scr):
    f32 = jnp.float32
    bf16 = jnp.bfloat16
    b = pl.program_id(0)
    i = pl.program_id(1)
    t0 = i * TT

    @pl.when((b == 0) & (i == 0))
    def _():
        wq_scr[:] = wqt_ref[:].astype(bf16)

    @pl.when(i == 0)
    def _():
        edu16 = edu_ref[0].astype(bf16)
        k_scr[:] = jax.lax.dot(edu16, wkt_ref[:].astype(bf16),
                               preferred_element_type=f32).astype(bf16)
        v_scr[:] = jax.lax.dot(edu16, wvt_ref[:].astype(bf16),
                               preferred_element_type=f32)

    tok = tok_ref[0]
    q = jax.lax.dot(tok.astype(bf16), wq_scr[:],
                    preferred_element_type=f32).astype(bf16)

    spk_row = spk_row_ref[0]
    spk_col = spk_col_ref[0]
    same = spk_col == spk_row
    trow = jax.lax.broadcasted_iota(jnp.int32, (TT, T), 0) + t0
    ucol = jax.lax.broadcasted_iota(jnp.int32, (TT, T), 1)
    mask = (same & (ucol <= trow)).astype(f32)
    mask = mask.reshape(TT, 1, T)

    parts = []
    for h in range(H):
        sl = slice(h * dh, (h + 1) * dh)
        q_h = q[:, sl]
        k_h = k_scr[:, sl]
        v_h = v_scr[:, sl]
        s = jax.lax.dot_general(q_h, k_h, (((1,), (1,)), ((), ())),
                                preferred_element_type=f32)
        s = s.reshape(TT, L, T) * mask
        a_h = jax.lax.dot(s.reshape(TT * L, T), v_h,
                          preferred_element_type=f32)
        parts.append(a_h)
    out_ref[0] = tok + jnp.concatenate(parts, axis=1)


def kernel(input_ids, speaker_names, token_embeddings, edu_embeddings,
           Wk, Wv, Wq):
    tok = token_embeddings.reshape(B, T * L, D)
    spk = speaker_names.astype(jnp.int32)
    spk_row = spk.reshape(B, 1, T)
    spk_col = spk.reshape(B, T, 1)

    out = pl.pallas_call(
        _body,
        grid=(B, NT),
        in_specs=[
            pl.BlockSpec((1, 1, T), lambda b, i: (b, 0, 0)),
            pl.BlockSpec((1, TT, 1), lambda b, i: (b, i, 0)),
            pl.BlockSpec((1, TT * L, D), lambda b, i: (b, i, 0)),
            pl.BlockSpec((1, T, D), lambda b, i: (b, 0, 0)),
            pl.BlockSpec((D, D), lambda b, i: (0, 0)),
            pl.BlockSpec((D, D), lambda b, i: (0, 0)),
            pl.BlockSpec((D, D), lambda b, i: (0, 0)),
        ],
        out_specs=pl.BlockSpec((1, TT * L, D), lambda b, i: (b, i, 0)),
        out_shape=jax.ShapeDtypeStruct((B, T * L, D), jnp.float32),
        scratch_shapes=[
            pltpu.VMEM((D, D), jnp.bfloat16),
            pltpu.VMEM((T, D), jnp.bfloat16),
            pltpu.VMEM((T, D), jnp.float32),
        ],
    )(spk_row, spk_col, tok, edu_embeddings, Wq.T, Wk.T, Wv.T)
    return out.reshape(B, T, L, D)


# speedup vs baseline: 1.1602x; 1.0819x over previous
import jax
import jax.numpy as jnp
from jax.experimental import pallas as pl
from jax.experimental.pallas import tpu as pltpu

B, T, L = 8, 32, 64
D = 768
H = 12
dh = D // H
G = 4
NG = H // G
TT = 16
NT = T // TT


def _body(spk4_ref, spk_col_ref, tok_ref, edu_ref, wqt_ref, wkt_ref,
          wvt_ref, out_ref, wq_scr, k4_scr, v4_scr):
    f32 = jnp.float32
    bf16 = jnp.bfloat16
    b = pl.program_id(0)
    i = pl.program_id(1)
    t0 = i * TT

    @pl.when((b == 0) & (i == 0))
    def _():
        wq_scr[:] = wqt_ref[:].astype(bf16)
        k4_scr[:] = jnp.zeros((G * D // G, G * T), bf16)
        v4_scr[:] = jnp.zeros((NG * G * T, G * dh), bf16)

    @pl.when(i == 0)
    def _():
        edu16 = edu_ref[0].astype(bf16)
        kT = jax.lax.dot_general(wkt_ref[:].astype(bf16), edu16,
                                 (((0,), (1,)), ((), ())),
                                 preferred_element_type=f32).astype(bf16)
        v16 = jax.lax.dot(edu16, wvt_ref[:].astype(bf16),
                          preferred_element_type=f32).astype(bf16)
        for h in range(H):
            j, r = divmod(h, G)
            k4_scr[G * dh * j + dh * r:G * dh * j + dh * (r + 1),
                   T * r:T * (r + 1)] = kT[dh * h:dh * (h + 1), :]
            v4_scr[G * T * j + T * r:G * T * j + T * (r + 1),
                   dh * r:dh * (r + 1)] = v16[:, dh * h:dh * (h + 1)]

    tok = tok_ref[0]
    q = jax.lax.dot(tok.astype(bf16), wq_scr[:],
                    preferred_element_type=f32).astype(bf16)

    spk4 = spk4_ref[0]
    spk_col = spk_col_ref[0]
    same = spk_col == spk4
    trow = jax.lax.broadcasted_iota(jnp.int32, (TT, G * T), 0) + t0
    ucol = jax.lax.broadcasted_iota(jnp.int32, (TT, G * T), 1) & (T - 1)
    mask4 = (same & (ucol <= trow)).astype(f32).reshape(TT, 1, G * T)

    for j in range(NG):
        csl = slice(G * dh * j, G * dh * (j + 1))
        s = jax.lax.dot(q[:, csl], k4_scr[csl, :],
                        preferred_element_type=f32)
        s = (s.reshape(TT, L, G * T) * mask4).astype(bf16)
        a_j = jax.lax.dot(s.reshape(TT * L, G * T),
                          v4_scr[G * T * j:G * T * (j + 1), :],
                          preferred_element_type=f32)
        out_ref[0, :, csl] = tok[:, csl] + a_j


def kernel(input_ids, speaker_names, token_embeddings, edu_embeddings,
           Wk, Wv, Wq):
    tok = token_embeddings.reshape(B, T * L, D)
    spk = speaker_names.astype(jnp.int32)
    spk4 = jnp.tile(spk.reshape(B, 1, T), (1, 1, G))
    spk_col = spk.reshape(B, T, 1)

    out = pl.pallas_call(
        _body,
        grid=(B, NT),
        in_specs=[
            pl.BlockSpec((1, 1, G * T), lambda b, i: (b, 0, 0)),
            pl.BlockSpec((1, TT, 1), lambda b, i: (b, i, 0)),
            pl.BlockSpec((1, TT * L, D), lambda b, i: (b, i, 0)),
            pl.BlockSpec((1, T, D), lambda b, i: (b, 0, 0)),
            pl.BlockSpec((D, D), lambda b, i: (0, 0)),
            pl.BlockSpec((D, D), lambda b, i: (0, 0)),
            pl.BlockSpec((D, D), lambda b, i: (0, 0)),
        ],
        out_specs=pl.BlockSpec((1, TT * L, D), lambda b, i: (b, i, 0)),
        out_shape=jax.ShapeDtypeStruct((B, T * L, D), jnp.float32),
        scratch_shapes=[
            pltpu.VMEM((D, D), jnp.bfloat16),
            pltpu.VMEM((D, G * T), jnp.bfloat16),
            pltpu.VMEM((NG * G * T, G * dh), jnp.bfloat16),
        ],
    )(spk4, spk_col, tok, edu_embeddings, Wq.T, Wk.T, Wv.T)
    return out.reshape(B, T, L, D)


# speedup vs baseline: 1.2048x; 1.0385x over previous
import jax
import jax.numpy as jnp
from jax.experimental import pallas as pl
from jax.experimental.pallas import tpu as pltpu

B, T, L = 8, 32, 64
D = 768
H = 12
dh = D // H
G = 4
NG = H // G
TT = 16
NT = T // TT


def _body(spk4_ref, spk_col_ref, tok_ref, edu_ref, wqt_ref, wkt_ref,
          wvt_ref, out_ref, k4_scr, v4_scr):
    f32 = jnp.float32
    bf16 = jnp.bfloat16
    b = pl.program_id(0)
    i = pl.program_id(1)
    t0 = i * TT

    @pl.when((b == 0) & (i == 0))
    def _():
        k4_scr[:] = jnp.zeros((G * D // G, G * T), bf16)
        v4_scr[:] = jnp.zeros((NG * G * T, G * dh), bf16)

    @pl.when(i == 0)
    def _():
        edu16 = edu_ref[0].astype(bf16)
        kT = jax.lax.dot_general(wkt_ref[:], edu16,
                                 (((0,), (1,)), ((), ())),
                                 preferred_element_type=f32).astype(bf16)
        v16 = jax.lax.dot(edu16, wvt_ref[:],
                          preferred_element_type=f32).astype(bf16)
        for h in range(H):
            j, r = divmod(h, G)
            k4_scr[G * dh * j + dh * r:G * dh * j + dh * (r + 1),
                   T * r:T * (r + 1)] = kT[dh * h:dh * (h + 1), :]
            v4_scr[G * T * j + T * r:G * T * j + T * (r + 1),
                   dh * r:dh * (r + 1)] = v16[:, dh * h:dh * (h + 1)]

    tok = tok_ref[0]
    q = jax.lax.dot(tok.astype(bf16), wqt_ref[:],
                    preferred_element_type=f32).astype(bf16)

    spk4 = spk4_ref[0]
    spk_col = spk_col_ref[0]
    same = spk_col == spk4
    trow = jax.lax.broadcasted_iota(jnp.int32, (TT, G * T), 0) + t0
    ucol = jax.lax.broadcasted_iota(jnp.int32, (TT, G * T), 1) & (T - 1)
    mask4 = (same & (ucol <= trow)).astype(f32).reshape(TT, 1, G * T)

    for j in range(NG):
        csl = slice(G * dh * j, G * dh * (j + 1))
        s = jax.lax.dot(q[:, csl], k4_scr[csl, :],
                        preferred_element_type=f32)
        s = (s.reshape(TT, L, G * T) * mask4).astype(bf16)
        a_j = jax.lax.dot(s.reshape(TT * L, G * T),
                          v4_scr[G * T * j:G * T * (j + 1), :],
                          preferred_element_type=f32)
        out_ref[0, :, csl] = tok[:, csl] + a_j


def kernel(input_ids, speaker_names, token_embeddings, edu_embeddings,
           Wk, Wv, Wq):
    tok = token_embeddings.reshape(B, T * L, D)
    spk = speaker_names.astype(jnp.int32)
    spk4 = jnp.tile(spk.reshape(B, 1, T), (1, 1, G))
    spk_col = spk.reshape(B, T, 1)

    out = pl.pallas_call(
        _body,
        grid=(B, NT),
        in_specs=[
            pl.BlockSpec((1, 1, G * T), lambda b, i: (b, 0, 0)),
            pl.BlockSpec((1, TT, 1), lambda b, i: (b, i, 0)),
            pl.BlockSpec((1, TT * L, D), lambda b, i: (b, i, 0)),
            pl.BlockSpec((1, T, D), lambda b, i: (b, 0, 0)),
            pl.BlockSpec((D, D), lambda b, i: (0, 0)),
            pl.BlockSpec((D, D), lambda b, i: (0, 0)),
            pl.BlockSpec((D, D), lambda b, i: (0, 0)),
        ],
        out_specs=pl.BlockSpec((1, TT * L, D), lambda b, i: (b, i, 0)),
        out_shape=jax.ShapeDtypeStruct((B, T * L, D), jnp.float32),
        scratch_shapes=[
            pltpu.VMEM((D, G * T), jnp.bfloat16),
            pltpu.VMEM((NG * G * T, G * dh), jnp.bfloat16),
        ],
    )(spk4, spk_col, tok, edu_embeddings,
      Wq.T.astype(jnp.bfloat16), Wk.T.astype(jnp.bfloat16),
      Wv.T.astype(jnp.bfloat16))
    return out.reshape(B, T, L, D)


# speedup vs baseline: 1.2054x; 1.0005x over previous
import jax
import jax.numpy as jnp
from jax.experimental import pallas as pl
from jax.experimental.pallas import tpu as pltpu

B, T, L = 8, 32, 64
D = 768
H = 12
dh = D // H
G = 4
NG = H // G
TT = 16
NT = T // TT


def _body(spk4_ref, spk_col_ref, tok_ref, edu_ref, wqt_ref, wkt_ref,
          wvt_ref, out_ref, k4_scr, v4_scr):
    f32 = jnp.float32
    bf16 = jnp.bfloat16
    b = pl.program_id(0)
    i = pl.program_id(1)
    t0 = i * TT

    @pl.when((b == 0) & (i == 0))
    def _():
        k4_scr[:] = jnp.zeros((G * D // G, G * T), bf16)
        v4_scr[:] = jnp.zeros((NG * G * T, G * dh), bf16)

    @pl.when(i == 0)
    def _():
        edu16 = edu_ref[0].astype(bf16)
        kT = jax.lax.dot_general(wkt_ref[:], edu16,
                                 (((0,), (1,)), ((), ())),
                                 preferred_element_type=f32).astype(bf16)
        v16 = jax.lax.dot(edu16, wvt_ref[:],
                          preferred_element_type=f32).astype(bf16)
        for h in range(H):
            j, r = divmod(h, G)
            k4_scr[G * dh * j + dh * r:G * dh * j + dh * (r + 1),
                   T * r:T * (r + 1)] = kT[dh * h:dh * (h + 1), :]
            v4_scr[G * T * j + T * r:G * T * j + T * (r + 1),
                   dh * r:dh * (r + 1)] = v16[:, dh * h:dh * (h + 1)]

    tok = tok_ref[0]
    q = jax.lax.dot(tok.astype(bf16), wqt_ref[:],
                    preferred_element_type=f32).astype(bf16)

    spk4 = spk4_ref[0]
    spk_col = spk_col_ref[0]
    same = spk_col == spk4
    trow = jax.lax.broadcasted_iota(jnp.int32, (TT, G * T), 0) + t0
    ucol = jax.lax.broadcasted_iota(jnp.int32, (TT, G * T), 1) & (T - 1)
    mask4 = (same & (ucol <= trow)).astype(f32).reshape(TT, 1, G * T)

    for j in range(NG):
        csl = slice(G * dh * j, G * dh * (j + 1))
        s = jax.lax.dot(q[:, csl], k4_scr[csl, :],
                        preferred_element_type=f32)
        s = (s.reshape(TT, L, G * T) * mask4).astype(bf16)
        a_j = jax.lax.dot(s.reshape(TT * L, G * T),
                          v4_scr[G * T * j:G * T * (j + 1), :],
                          preferred_element_type=f32)
        out_ref[0, :, csl] = tok[:, csl] + a_j


def kernel(input_ids, speaker_names, token_embeddings, edu_embeddings,
           Wk, Wv, Wq):
    tok = token_embeddings.reshape(B, T * L, D)
    spk = speaker_names.astype(jnp.int32)
    spk4 = jnp.tile(spk.reshape(B, 1, T), (1, 1, G))
    spk_col = spk.reshape(B, T, 1)

    out = pl.pallas_call(
        _body,
        grid=(B, NT),
        in_specs=[
            pl.BlockSpec((1, 1, G * T), lambda b, i: (b, 0, 0)),
            pl.BlockSpec((1, TT, 1), lambda b, i: (b, i, 0)),
            pl.BlockSpec((1, TT * L, D), lambda b, i: (b, i, 0)),
            pl.BlockSpec((1, T, D), lambda b, i: (b, 0, 0)),
            pl.BlockSpec((D, D), lambda b, i: (0, 0)),
            pl.BlockSpec((D, D), lambda b, i: (0, 0)),
            pl.BlockSpec((D, D), lambda b, i: (0, 0)),
        ],
        out_specs=pl.BlockSpec((1, TT * L, D), lambda b, i: (b, i, 0)),
        out_shape=jax.ShapeDtypeStruct((B, T * L, D), jnp.float32),
        compiler_params=pltpu.CompilerParams(
            vmem_limit_bytes=60 * 1024 * 1024),
        scratch_shapes=[
            pltpu.VMEM((D, G * T), jnp.bfloat16),
            pltpu.VMEM((NG * G * T, G * dh), jnp.bfloat16),
        ],
    )(spk4, spk_col, tok, edu_embeddings,
      Wq.T.astype(jnp.bfloat16), Wk.T.astype(jnp.bfloat16),
      Wv.T.astype(jnp.bfloat16))
    return out.reshape(B, T, L, D)
